# Initial kernel scaffold; baseline (speedup 1.0000x reference)
import jax
import jax.numpy as jnp
from jax.experimental import pallas as pl


def kernel(x, edge_index, edge_weight, W0, b0, W1, b1):
    raise NotImplementedError("write your pallas kernel here")


# speedup vs baseline: 1.4436x; 1.4436x over previous
import functools

import jax
import jax.numpy as jnp
from jax.experimental import pallas as pl

N = 10000
E = 160000
D = 256
K = 3
H = 64
C = 40

ROW_BLK = 512


def _matmul_relu_body(nk, relu, h_ref, w_ref, b_ref, o_ref):
    acc = jnp.zeros((ROW_BLK, o_ref.shape[-1]), jnp.float32)
    for k in range(nk):
        acc += jnp.dot(h_ref[k], w_ref[k], preferred_element_type=jnp.float32)
    acc += b_ref[...][None, :]
    if relu:
        acc = jnp.maximum(acc, 0.0)
    o_ref[...] = acc


def _fused_matmul(hops, w, b, relu):
    kh, np_, f = hops.shape
    o = w.shape[-1]
    grid = (np_ // ROW_BLK,)
    return pl.pallas_call(
        functools.partial(_matmul_relu_body, kh, relu),
        grid=grid,
        in_specs=[
            pl.BlockSpec((kh, ROW_BLK, f), lambda i: (0, i, 0)),
            pl.BlockSpec((kh, f, o), lambda i: (0, 0, 0)),
            pl.BlockSpec((o,), lambda i: (0,)),
        ],
        out_specs=pl.BlockSpec((ROW_BLK, o), lambda i: (i, 0)),
        out_shape=jax.ShapeDtypeStruct((np_, o), jnp.float32),
    )(hops, w, b)


def _prop(h, src, dst, norm_w, selfw):
    feats = [h]
    cur = h
    for _ in range(K):
        cur = (selfw[:, None] * cur).at[dst].add(norm_w[:, None] * cur[src])
        feats.append(cur)
    return jnp.stack(feats, axis=0)


def kernel(x, edge_index, edge_weight, W0, b0, W1, b1):
    src = edge_index[0]
    dst = edge_index[1]
    deg = jnp.ones((N,), jnp.float32).at[dst].add(edge_weight)
    deg = jnp.maximum(deg, 1e-12)
    inv_sqrt = jax.lax.rsqrt(deg)
    selfw = 1.0 / deg
    norm_w = edge_weight * inv_sqrt[src] * inv_sqrt[dst]

    npad = 10240
    xp = jnp.pad(x, ((0, npad - N), (0, 0)))
    selfw_p = jnp.pad(selfw, (0, npad - N))

    hops0 = _prop(xp, src, dst, norm_w, selfw_p)
    w0 = W0.reshape(K + 1, D, H)
    h1 = _fused_matmul(hops0, w0, b0, relu=True)

    hops1 = _prop(h1, src, dst, norm_w, selfw_p)
    w1 = W1.reshape(K + 1, H, C)
    out = _fused_matmul(hops1, w1, b1, relu=False)
    return out[:N]


# speedup vs baseline: 3.5870x; 2.4848x over previous
import dataclasses
import functools

import jax
import jax.numpy as jnp
from jax import lax
from jax.experimental import pallas as pl
from jax.experimental.pallas import tpu as pltpu
from jax.experimental.pallas import tpu_sc as plsc

N = 10000
E = 160000
D = 256
K = 3
H = 64
C = 40

NP = 10240
B = 128
NBLK = 40
NW = 32
EP = NW * NBLK * B
L = 16
RPT = NP // 16

ROW_BLK = 512

_MESH = plsc.VectorSubcoreMesh(core_axis_name="c", subcore_axis_name="s")

_SC_PARAMS = pltpu.CompilerParams()
if "needs_layout_passes" in pltpu.CompilerParams.__dataclass_fields__:
    _SC_PARAMS = dataclasses.replace(_SC_PARAMS, needs_layout_passes=False)


def _zero_rows(rows_ref, nrow, ncol):
    zeros = jnp.zeros((L,), jnp.float32)

    @pl.loop(0, nrow)
    def _(b):
        for f0 in range(0, ncol, L):
            rows_ref[b, pl.ds(f0, L)] = zeros


def _deg_body(dst_hbm, w_hbm, out_hbm, dst_v, w_v, rows_v, acc_sh):
    c = lax.axis_index("c")
    s = lax.axis_index("s")
    wid = c * 16 + s
    row0 = s * RPT
    pltpu.sync_copy(dst_hbm.at[wid], dst_v)
    pltpu.sync_copy(w_hbm.at[wid], w_v)
    _zero_rows(rows_v, B, L)
    for kk in range(RPT // B):
        pltpu.sync_copy(rows_v, acc_sh.at[pl.ds(row0 + kk * B, B)])
    plsc.subcore_barrier()

    zcol = jnp.zeros((L,), jnp.int32)

    @pl.loop(0, NBLK)
    def _(j):
        for b0 in range(0, B, L):
            wv = w_v[j, pl.ds(b0, L)]
            ridx = b0 + lax.iota(jnp.int32, L)
            plsc.store_scatter(rows_v, [ridx, zcol], wv)
        pltpu.sync_copy(rows_v, acc_sh.at[dst_v.at[j]], add=True)

    plsc.subcore_barrier()
    pltpu.sync_copy(acc_sh.at[pl.ds(row0, RPT)],
                    out_hbm.at[c, pl.ds(row0, RPT)])


@functools.partial(jax.jit)
def _deg_call(dstp, wp):
    f = pl.kernel(
        _deg_body,
        out_type=jax.ShapeDtypeStruct((2, NP, L), jnp.float32),
        mesh=_MESH,
        compiler_params=_SC_PARAMS,
        scratch_types=[
            pltpu.VMEM((NBLK, B), jnp.int32),
            pltpu.VMEM((NBLK, B), jnp.float32),
            pltpu.VMEM((B, L), jnp.float32),
            pltpu.VMEM_SHARED((NP, L), jnp.float32),
        ],
    )
    return f(dstp, wp)


def _normw_body(invs_hbm, src_hbm, dst_hbm, w_hbm, out_hbm,
                invs_v, src_v, dst_v, w_v, nw_v):
    c = lax.axis_index("c")
    s = lax.axis_index("s")
    wid = c * 16 + s
    pltpu.sync_copy(invs_hbm, invs_v)
    pltpu.sync_copy(src_hbm.at[wid], src_v)
    pltpu.sync_copy(dst_hbm.at[wid], dst_v)
    pltpu.sync_copy(w_hbm.at[wid], w_v)

    @pl.loop(0, NBLK)
    def _(j):
        for b0 in range(0, B, L):
            sv = plsc.load_gather(invs_v, [src_v[j, pl.ds(b0, L)]])
            dv = plsc.load_gather(invs_v, [dst_v[j, pl.ds(b0, L)]])
            nw_v[j, pl.ds(b0, L)] = w_v[j, pl.ds(b0, L)] * sv * dv

    pltpu.sync_copy(nw_v, out_hbm.at[wid])


@functools.partial(jax.jit)
def _normw_call(invs, srcp, dstp, wp):
    f = pl.kernel(
        _normw_body,
        out_type=jax.ShapeDtypeStruct((NW, NBLK, B), jnp.float32),
        mesh=_MESH,
        compiler_params=_SC_PARAMS,
        scratch_types=[
            pltpu.VMEM((NP,), jnp.float32),
            pltpu.VMEM((NBLK, B), jnp.int32),
            pltpu.VMEM((NBLK, B), jnp.int32),
            pltpu.VMEM((NBLK, B), jnp.float32),
            pltpu.VMEM((NBLK, B), jnp.float32),
        ],
    )
    return f(invs, srcp, dstp, wp)


def _make_hop(nck, f):
    def body(h_hbm, src_hbm, dst_hbm, w_hbm, out_hbm,
             src_v, dst_v, w_v, rows_v, acc_sh):
        c = lax.axis_index("c")
        s = lax.axis_index("s")
        wid = c * 16 + s
        row0 = s * RPT
        pltpu.sync_copy(src_hbm.at[wid], src_v)
        pltpu.sync_copy(dst_hbm.at[wid], dst_v)
        pltpu.sync_copy(w_hbm.at[wid], w_v)
        for ci in range(nck):
            _zero_rows(rows_v, B, f)
            for kk in range(RPT // B):
                pltpu.sync_copy(rows_v, acc_sh.at[pl.ds(row0 + kk * B, B)])
            plsc.subcore_barrier()

            @pl.loop(0, NBLK)
            def _(j):
                pltpu.sync_copy(h_hbm.at[ci].at[src_v.at[j]], rows_v)
                jv = jnp.full((L,), j, jnp.int32)

                @pl.loop(0, B)
                def _(b):
                    wb = plsc.load_gather(
                        w_v, [jv, jnp.full((L,), b, jnp.int32)])
                    for f0 in range(0, f, L):
                        rows_v[b, pl.ds(f0, L)] = rows_v[b, pl.ds(f0, L)] * wb

                pltpu.sync_copy(rows_v, acc_sh.at[dst_v.at[j]], add=True)

            plsc.subcore_barrier()
            pltpu.sync_copy(acc_sh.at[pl.ds(row0, RPT)],
                            out_hbm.at[c, ci, pl.ds(row0, RPT)])

    def call(h2, srcp, dstp, nwp):
        fn = pl.kernel(
            body,
            out_type=jax.ShapeDtypeStruct((2, nck, NP, f), jnp.float32),
            mesh=_MESH,
            compiler_params=_SC_PARAMS,
            scratch_types=[
                pltpu.VMEM((NBLK, B), jnp.int32),
                pltpu.VMEM((NBLK, B), jnp.int32),
                pltpu.VMEM((NBLK, B), jnp.float32),
                pltpu.VMEM((B, f), jnp.float32),
                pltpu.VMEM_SHARED((NP, f), jnp.float32),
            ],
        )
        return fn(h2, srcp, dstp, nwp)

    return call


_hop_l1 = _make_hop(2, 128)
_hop_l2 = _make_hop(1, 128)


def _finalize_body(p_ref, invs_ref, selfw_ref):
    d = 1.0 + p_ref[0] + p_ref[1]
    d = jnp.maximum(d, 1e-12)
    invs_ref[...] = lax.rsqrt(d)[:, :1]
    selfw_ref[...] = (1.0 / d)[:, :1]


def _finalize(dpart):
    return pl.pallas_call(
        _finalize_body,
        grid=(NP // ROW_BLK,),
        in_specs=[pl.BlockSpec((2, ROW_BLK, L), lambda i: (0, i, 0))],
        out_specs=[pl.BlockSpec((ROW_BLK, 1), lambda i: (i, 0)),
                   pl.BlockSpec((ROW_BLK, 1), lambda i: (i, 0))],
        out_shape=[jax.ShapeDtypeStruct((NP, 1), jnp.float32),
                   jax.ShapeDtypeStruct((NP, 1), jnp.float32)],
    )(dpart)


def _combine_body(nck, p_ref, h_ref, sw_ref, o_ref):
    sw = sw_ref[...]
    for ck in range(nck):
        o_ref[ck] = p_ref[0, ck] + p_ref[1, ck] + sw * h_ref[ck]


def _combine(parts, h2, selfw):
    nck, _, f = h2.shape
    return pl.pallas_call(
        functools.partial(_combine_body, nck),
        grid=(NP // ROW_BLK,),
        in_specs=[
            pl.BlockSpec((2, nck, ROW_BLK, f), lambda i: (0, 0, i, 0)),
            pl.BlockSpec((nck, ROW_BLK, f), lambda i: (0, i, 0)),
            pl.BlockSpec((ROW_BLK, 1), lambda i: (i, 0)),
        ],
        out_specs=pl.BlockSpec((nck, ROW_BLK, f), lambda i: (0, i, 0)),
        out_shape=jax.ShapeDtypeStruct((nck, NP, f), jnp.float32),
    )(parts, h2, selfw)


def _mm_body(nck, relu, h0_ref, h1_ref, h2_ref, h3_ref, w_ref, b_ref, o_ref):
    o = o_ref.shape[-1]
    acc = jnp.zeros((ROW_BLK, o), jnp.float32) + b_ref[...][None, :]
    for k, href in enumerate((h0_ref, h1_ref, h2_ref, h3_ref)):
        for ck in range(nck):
            acc += jnp.dot(href[ck], w_ref[k, ck],
                           preferred_element_type=jnp.float32)
    if relu:
        acc = jnp.maximum(acc, 0.0)
    o_ref[...] = acc


def _mm(hops, w, b, relu):
    nck, _, f = hops[0].shape
    o = w.shape[-1]
    hspec = pl.BlockSpec((nck, ROW_BLK, f), lambda i: (0, i, 0))
    return pl.pallas_call(
        functools.partial(_mm_body, nck, relu),
        grid=(NP // ROW_BLK,),
        in_specs=[hspec, hspec, hspec, hspec,
                  pl.BlockSpec((4, nck, f, o), lambda i: (0, 0, 0, 0)),
                  pl.BlockSpec((o,), lambda i: (0,))],
        out_specs=pl.BlockSpec((ROW_BLK, o), lambda i: (i, 0)),
        out_shape=jax.ShapeDtypeStruct((NP, o), jnp.float32),
    )(*hops, w, b)


def kernel(x, edge_index, edge_weight, W0, b0, W1, b1):
    src = edge_index[0]
    dst = edge_index[1]
    pad = EP - E
    srcp = jnp.pad(src, (0, pad)).reshape(NW, NBLK, B)
    dstp = jnp.pad(dst, (0, pad)).reshape(NW, NBLK, B)
    wp = jnp.pad(edge_weight, (0, pad)).reshape(NW, NBLK, B)

    dpart = _deg_call(dstp, wp)
    invs1, selfw = _finalize(dpart)
    nwp = _normw_call(invs1.reshape(NP), srcp, dstp, wp)

    xp = jnp.pad(x, ((0, NP - N), (0, 0)))
    h2 = xp.reshape(NP, 2, 128).transpose(1, 0, 2)
    hops1 = [h2]
    for _ in range(K):
        parts = _hop_l1(h2, srcp, dstp, nwp)
        h2 = _combine(parts, h2, selfw)
        hops1.append(h2)
    h1 = _mm(hops1, W0.reshape(K + 1, 2, 128, H), b0, relu=True)

    g = jnp.pad(h1, ((0, 0), (0, 128 - H))).reshape(1, NP, 128)
    hops2 = [g]
    for _ in range(K):
        parts = _hop_l2(g, srcp, dstp, nwp)
        g = _combine(parts, g, selfw)
        hops2.append(g)
    w1p = jnp.pad(W1.reshape(K + 1, H, C), ((0, 0), (0, 128 - H), (0, 0)))
    out = _mm(hops2, w1p.reshape(K + 1, 1, 128, C), b1, relu=False)
    return out[:N]


# speedup vs baseline: 3.7090x; 1.0340x over previous
import dataclasses
import functools

import jax
import jax.numpy as jnp
from jax import lax
from jax.experimental import pallas as pl
from jax.experimental.pallas import tpu as pltpu
from jax.experimental.pallas import tpu_sc as plsc

N = 10000
E = 160000
D = 256
K = 3
H = 64
C = 40

NP = 10240
B = 128
NBLK = 40
NW = 32
EP = NW * NBLK * B
L = 16
RPT = NP // 16

ROW_BLK = 512

_MESH = plsc.VectorSubcoreMesh(core_axis_name="c", subcore_axis_name="s")

_SC_PARAMS = pltpu.CompilerParams()
if "needs_layout_passes" in pltpu.CompilerParams.__dataclass_fields__:
    _SC_PARAMS = dataclasses.replace(_SC_PARAMS, needs_layout_passes=False)


def _zero_rows(rows_ref, nrow, ncol):
    zeros = jnp.zeros((L,), jnp.float32)

    @pl.loop(0, nrow)
    def _(b):
        for f0 in range(0, ncol, L):
            rows_ref[b, pl.ds(f0, L)] = zeros


def _deg_body(dst_hbm, w_hbm, out_hbm, dst_v, w_v, rows_v, acc_sh):
    c = lax.axis_index("c")
    s = lax.axis_index("s")
    wid = c * 16 + s
    row0 = s * RPT
    pltpu.sync_copy(dst_hbm.at[wid], dst_v)
    pltpu.sync_copy(w_hbm.at[wid], w_v)
    _zero_rows(rows_v, B, L)
    for kk in range(RPT // B):
        pltpu.sync_copy(rows_v, acc_sh.at[pl.ds(row0 + kk * B, B)])
    plsc.subcore_barrier()

    zcol = jnp.zeros((L,), jnp.int32)

    @pl.loop(0, NBLK)
    def _(j):
        for b0 in range(0, B, L):
            wv = w_v[j, pl.ds(b0, L)]
            ridx = b0 + lax.iota(jnp.int32, L)
            plsc.store_scatter(rows_v, [ridx, zcol], wv)
        pltpu.sync_copy(rows_v, acc_sh.at[dst_v.at[j]], add=True)

    plsc.subcore_barrier()
    pltpu.sync_copy(acc_sh.at[pl.ds(row0, RPT)],
                    out_hbm.at[c, pl.ds(row0, RPT)])


@functools.partial(jax.jit)
def _deg_call(dstp, wp):
    f = pl.kernel(
        _deg_body,
        out_type=jax.ShapeDtypeStruct((2, NP, L), jnp.float32),
        mesh=_MESH,
        compiler_params=_SC_PARAMS,
        scratch_types=[
            pltpu.VMEM((NBLK, B), jnp.int32),
            pltpu.VMEM((NBLK, B), jnp.float32),
            pltpu.VMEM((B, L), jnp.float32),
            pltpu.VMEM_SHARED((NP, L), jnp.float32),
        ],
    )
    return f(dstp, wp)


def _normw_body(invs_hbm, src_hbm, dst_hbm, w_hbm, out_hbm,
                invs_v, src_v, dst_v, w_v, nw_v):
    c = lax.axis_index("c")
    s = lax.axis_index("s")
    wid = c * 16 + s
    pltpu.sync_copy(invs_hbm, invs_v)
    pltpu.sync_copy(src_hbm.at[wid], src_v)
    pltpu.sync_copy(dst_hbm.at[wid], dst_v)
    pltpu.sync_copy(w_hbm.at[wid], w_v)

    @pl.loop(0, NBLK)
    def _(j):
        for b0 in range(0, B, L):
            sv = plsc.load_gather(invs_v, [src_v[j, pl.ds(b0, L)]])
            dv = plsc.load_gather(invs_v, [dst_v[j, pl.ds(b0, L)]])
            nw_v[j, pl.ds(b0, L)] = w_v[j, pl.ds(b0, L)] * sv * dv

    pltpu.sync_copy(nw_v, out_hbm.at[wid])


@functools.partial(jax.jit)
def _normw_call(invs, srcp, dstp, wp):
    f = pl.kernel(
        _normw_body,
        out_type=jax.ShapeDtypeStruct((NW, NBLK, B), jnp.float32),
        mesh=_MESH,
        compiler_params=_SC_PARAMS,
        scratch_types=[
            pltpu.VMEM((NP,), jnp.float32),
            pltpu.VMEM((NBLK, B), jnp.int32),
            pltpu.VMEM((NBLK, B), jnp.int32),
            pltpu.VMEM((NBLK, B), jnp.float32),
            pltpu.VMEM((NBLK, B), jnp.float32),
        ],
    )
    return f(invs, srcp, dstp, wp)


_SPLAT_DNUMS = lax.GatherDimensionNumbers(
    offset_dims=(), collapsed_slice_dims=(0,), start_index_map=(0,))


def _splat(vec, i):
    idx = jnp.full((L, 1), i, jnp.int32)
    return lax.gather(vec, idx, _SPLAT_DNUMS, (1,),
                      mode=lax.GatherScatterMode.PROMISE_IN_BOUNDS)


BH = 64
NBLKH = EP // NW // BH


def _make_hop(nck, f):
    def scale(rows_ref, w_v, j):
        @pl.loop(0, BH, step=L)
        def _(b0):
            wv = w_v[j, pl.ds(b0, L)]
            for r in range(L):
                ws = _splat(wv, r)
                for f0 in range(0, f, L):
                    rows_ref[b0 + r, pl.ds(f0, L)] = (
                        rows_ref[b0 + r, pl.ds(f0, L)] * ws)

    def body(h_hbm, src_hbm, dst_hbm, w_hbm, out_hbm,
             src_v, dst_v, w_v, rows_a, rows_b, sem_a, sem_b,
             acc_sh):
        c = lax.axis_index("c")
        s = lax.axis_index("s")
        wid = c * 16 + s
        row0 = s * RPT
        pltpu.sync_copy(src_hbm.at[wid], src_v)
        pltpu.sync_copy(dst_hbm.at[wid], dst_v)
        pltpu.sync_copy(w_hbm.at[wid], w_v)
        for ci in range(nck):
            _zero_rows(rows_a, BH, f)
            for kk in range(RPT // BH):
                pltpu.sync_copy(rows_a, acc_sh.at[pl.ds(row0 + kk * BH, BH)])
            plsc.subcore_barrier()

            @pl.loop(0, NBLKH)
            def _(j):
                pltpu.sync_copy(h_hbm.at[ci].at[src_v.at[j]], rows_a)
                scale(rows_a, w_v, j)
                pltpu.sync_copy(rows_a, acc_sh.at[dst_v.at[j]], add=True)

            plsc.subcore_barrier()
            pltpu.sync_copy(acc_sh.at[pl.ds(row0, RPT)],
                            out_hbm.at[c, ci, pl.ds(row0, RPT)])

    def call(h2, srcp, dstp, nwp):
        fn = pl.kernel(
            body,
            out_type=jax.ShapeDtypeStruct((2, nck, NP, f), jnp.float32),
            mesh=_MESH,
            compiler_params=_SC_PARAMS,
            scratch_types=[
                pltpu.VMEM((NBLKH, BH), jnp.int32),
                pltpu.VMEM((NBLKH, BH), jnp.int32),
                pltpu.VMEM((NBLKH, BH), jnp.float32),
                pltpu.VMEM((BH, f), jnp.float32),
                pltpu.VMEM((BH, f), jnp.float32),
                pltpu.SemaphoreType.DMA,
                pltpu.SemaphoreType.DMA,
                pltpu.VMEM_SHARED((NP, f), jnp.float32),
            ],
        )
        return fn(h2, srcp.reshape(NW, NBLKH, BH),
                  dstp.reshape(NW, NBLKH, BH), nwp.reshape(NW, NBLKH, BH))

    return call


_hop_l1 = _make_hop(2, 128)
_hop_l2 = _make_hop(1, 128)


def _finalize_body(p_ref, invs_ref, selfw_ref):
    d = 1.0 + p_ref[0] + p_ref[1]
    d = jnp.maximum(d, 1e-12)
    invs_ref[...] = lax.rsqrt(d)[:, :1]
    selfw_ref[...] = (1.0 / d)[:, :1]


def _finalize(dpart):
    return pl.pallas_call(
        _finalize_body,
        grid=(NP // ROW_BLK,),
        in_specs=[pl.BlockSpec((2, ROW_BLK, L), lambda i: (0, i, 0))],
        out_specs=[pl.BlockSpec((ROW_BLK, 1), lambda i: (i, 0)),
                   pl.BlockSpec((ROW_BLK, 1), lambda i: (i, 0))],
        out_shape=[jax.ShapeDtypeStruct((NP, 1), jnp.float32),
                   jax.ShapeDtypeStruct((NP, 1), jnp.float32)],
    )(dpart)


def _combine_body(nck, p_ref, h_ref, sw_ref, o_ref):
    sw = sw_ref[...]
    for ck in range(nck):
        o_ref[ck] = p_ref[0, ck] + p_ref[1, ck] + sw * h_ref[ck]


def _combine(parts, h2, selfw):
    nck, _, f = h2.shape
    return pl.pallas_call(
        functools.partial(_combine_body, nck),
        grid=(NP // ROW_BLK,),
        in_specs=[
            pl.BlockSpec((2, nck, ROW_BLK, f), lambda i: (0, 0, i, 0)),
            pl.BlockSpec((nck, ROW_BLK, f), lambda i: (0, i, 0)),
            pl.BlockSpec((ROW_BLK, 1), lambda i: (i, 0)),
        ],
        out_specs=pl.BlockSpec((nck, ROW_BLK, f), lambda i: (0, i, 0)),
        out_shape=jax.ShapeDtypeStruct((nck, NP, f), jnp.float32),
    )(parts, h2, selfw)


def _mm_body(nck, relu, h0_ref, h1_ref, h2_ref, h3_ref, w_ref, b_ref, o_ref):
    o = o_ref.shape[-1]
    acc = jnp.zeros((ROW_BLK, o), jnp.float32) + b_ref[...][None, :]
    for k, href in enumerate((h0_ref, h1_ref, h2_ref, h3_ref)):
        for ck in range(nck):
            acc += jnp.dot(href[ck], w_ref[k, ck],
                           preferred_element_type=jnp.float32)
    if relu:
        acc = jnp.maximum(acc, 0.0)
    o_ref[...] = acc


def _mm(hops, w, b, relu):
    nck, _, f = hops[0].shape
    o = w.shape[-1]
    hspec = pl.BlockSpec((nck, ROW_BLK, f), lambda i: (0, i, 0))
    return pl.pallas_call(
        functools.partial(_mm_body, nck, relu),
        grid=(NP // ROW_BLK,),
        in_specs=[hspec, hspec, hspec, hspec,
                  pl.BlockSpec((4, nck, f, o), lambda i: (0, 0, 0, 0)),
                  pl.BlockSpec((o,), lambda i: (0,))],
        out_specs=pl.BlockSpec((ROW_BLK, o), lambda i: (i, 0)),
        out_shape=jax.ShapeDtypeStruct((NP, o), jnp.float32),
    )(*hops, w, b)


def kernel(x, edge_index, edge_weight, W0, b0, W1, b1):
    src = edge_index[0]
    dst = edge_index[1]
    pad = EP - E
    srcp = jnp.pad(src, (0, pad)).reshape(NW, NBLK, B)
    dstp = jnp.pad(dst, (0, pad)).reshape(NW, NBLK, B)
    wp = jnp.pad(edge_weight, (0, pad)).reshape(NW, NBLK, B)

    dpart = _deg_call(dstp, wp)
    invs1, selfw = _finalize(dpart)
    nwp = _normw_call(invs1.reshape(NP), srcp, dstp, wp)

    xp = jnp.pad(x, ((0, NP - N), (0, 0)))
    h2 = xp.reshape(NP, 2, 128).transpose(1, 0, 2)
    hops1 = [h2]
    for _ in range(K):
        parts = _hop_l1(h2, srcp, dstp, nwp)
        h2 = _combine(parts, h2, selfw)
        hops1.append(h2)
    h1 = _mm(hops1, W0.reshape(K + 1, 2, 128, H), b0, relu=True)

    g = jnp.pad(h1, ((0, 0), (0, 128 - H))).reshape(1, NP, 128)
    hops2 = [g]
    for _ in range(K):
        parts = _hop_l2(g, srcp, dstp, nwp)
        g = _combine(parts, g, selfw)
        hops2.append(g)
    w1p = jnp.pad(W1.reshape(K + 1, H, C), ((0, 0), (0, 128 - H), (0, 0)))
    out = _mm(hops2, w1p.reshape(K + 1, 1, 128, C), b1, relu=False)
    return out[:N]


# speedup vs baseline: 3.8644x; 1.0419x over previous
import dataclasses
import functools

import jax
import jax.numpy as jnp
from jax import lax
from jax.experimental import pallas as pl
from jax.experimental.pallas import tpu as pltpu
from jax.experimental.pallas import tpu_sc as plsc

N = 10000
E = 160000
D = 256
K = 3
H = 64
C = 40

NP = 10240
B = 128
NBLK = 40
NW = 32
EP = NW * NBLK * B
L = 16
RPT = NP // 16

ROW_BLK = 512

_MESH = plsc.VectorSubcoreMesh(core_axis_name="c", subcore_axis_name="s")

_SC_PARAMS = pltpu.CompilerParams()
if "needs_layout_passes" in pltpu.CompilerParams.__dataclass_fields__:
    _SC_PARAMS = dataclasses.replace(_SC_PARAMS, needs_layout_passes=False)


def _zero_rows(rows_ref, nrow, ncol):
    zeros = jnp.zeros((L,), jnp.float32)

    @pl.loop(0, nrow)
    def _(b):
        for f0 in range(0, ncol, L):
            rows_ref[b, pl.ds(f0, L)] = zeros


def _deg_body(dst_hbm, w_hbm, out_hbm, dst_v, w_v, rows_v, acc_sh):
    c = lax.axis_index("c")
    s = lax.axis_index("s")
    wid = c * 16 + s
    row0 = s * RPT
    pltpu.sync_copy(dst_hbm.at[wid], dst_v)
    pltpu.sync_copy(w_hbm.at[wid], w_v)
    _zero_rows(rows_v, B, L)
    for kk in range(RPT // B):
        pltpu.sync_copy(rows_v, acc_sh.at[pl.ds(row0 + kk * B, B)])
    plsc.subcore_barrier()

    zcol = jnp.zeros((L,), jnp.int32)

    @pl.loop(0, NBLK)
    def _(j):
        for b0 in range(0, B, L):
            wv = w_v[j, pl.ds(b0, L)]
            ridx = b0 + lax.iota(jnp.int32, L)
            plsc.store_scatter(rows_v, [ridx, zcol], wv)
        pltpu.sync_copy(rows_v, acc_sh.at[dst_v.at[j]], add=True)

    plsc.subcore_barrier()
    pltpu.sync_copy(acc_sh.at[pl.ds(row0, RPT)],
                    out_hbm.at[c, pl.ds(row0, RPT)])


@functools.partial(jax.jit)
def _deg_call(dstp, wp):
    f = pl.kernel(
        _deg_body,
        out_type=jax.ShapeDtypeStruct((2, NP, L), jnp.float32),
        mesh=_MESH,
        compiler_params=_SC_PARAMS,
        scratch_types=[
            pltpu.VMEM((NBLK, B), jnp.int32),
            pltpu.VMEM((NBLK, B), jnp.float32),
            pltpu.VMEM((B, L), jnp.float32),
            pltpu.VMEM_SHARED((NP, L), jnp.float32),
        ],
    )
    return f(dstp, wp)


def _normw_body(invs_hbm, src_hbm, dst_hbm, w_hbm, out_hbm,
                invs_v, src_v, dst_v, w_v, nw_v):
    c = lax.axis_index("c")
    s = lax.axis_index("s")
    wid = c * 16 + s
    pltpu.sync_copy(invs_hbm, invs_v)
    pltpu.sync_copy(src_hbm.at[wid], src_v)
    pltpu.sync_copy(dst_hbm.at[wid], dst_v)
    pltpu.sync_copy(w_hbm.at[wid], w_v)

    @pl.loop(0, NBLK)
    def _(j):
        for b0 in range(0, B, L):
            sv = plsc.load_gather(invs_v, [src_v[j, pl.ds(b0, L)]])
            dv = plsc.load_gather(invs_v, [dst_v[j, pl.ds(b0, L)]])
            nw_v[j, pl.ds(b0, L)] = w_v[j, pl.ds(b0, L)] * sv * dv

    pltpu.sync_copy(nw_v, out_hbm.at[wid])


@functools.partial(jax.jit)
def _normw_call(invs, srcp, dstp, wp):
    f = pl.kernel(
        _normw_body,
        out_type=jax.ShapeDtypeStruct((NW, NBLK, B), jnp.float32),
        mesh=_MESH,
        compiler_params=_SC_PARAMS,
        scratch_types=[
            pltpu.VMEM((NP,), jnp.float32),
            pltpu.VMEM((NBLK, B), jnp.int32),
            pltpu.VMEM((NBLK, B), jnp.int32),
            pltpu.VMEM((NBLK, B), jnp.float32),
            pltpu.VMEM((NBLK, B), jnp.float32),
        ],
    )
    return f(invs, srcp, dstp, wp)


_SPLAT_DNUMS = lax.GatherDimensionNumbers(
    offset_dims=(), collapsed_slice_dims=(0,), start_index_map=(0,))


def _splat(vec, i):
    idx = jnp.full((L, 1), i, jnp.int32)
    return lax.gather(vec, idx, _SPLAT_DNUMS, (1,),
                      mode=lax.GatherScatterMode.PROMISE_IN_BOUNDS)


BH = 128
NBLKH = EP // NW // BH


def _make_hop(nck, f):
    def scale(rows_ref, w_v, j):
        @pl.loop(0, BH, step=L)
        def _(b0):
            wv = w_v[j, pl.ds(b0, L)]
            for r in range(L):
                ws = _splat(wv, r)
                for f0 in range(0, f, L):
                    rows_ref[b0 + r, pl.ds(f0, L)] = (
                        rows_ref[b0 + r, pl.ds(f0, L)] * ws)

    def body(h_hbm, src_hbm, dst_hbm, w_hbm, out_hbm,
             src_v, dst_v, w_v, rows_a, acc_sh):
        c = lax.axis_index("c")
        s = lax.axis_index("s")
        wid = c * 16 + s
        row0 = s * RPT
        pltpu.sync_copy(src_hbm.at[wid], src_v)
        pltpu.sync_copy(dst_hbm.at[wid], dst_v)
        pltpu.sync_copy(w_hbm.at[wid], w_v)
        for ci in range(nck):
            _zero_rows(rows_a, BH, f)
            for kk in range(RPT // BH):
                pltpu.sync_copy(rows_a, acc_sh.at[pl.ds(row0 + kk * BH, BH)])
            plsc.subcore_barrier()

            @pl.loop(0, NBLKH)
            def _(j):
                pltpu.sync_copy(h_hbm.at[ci].at[src_v.at[j]], rows_a)
                scale(rows_a, w_v, j)
                pltpu.sync_copy(rows_a, acc_sh.at[dst_v.at[j]], add=True)

            plsc.subcore_barrier()
            pltpu.sync_copy(acc_sh.at[pl.ds(row0, RPT)],
                            out_hbm.at[c, ci, pl.ds(row0, RPT)])

    def call(h2, srcp, dstp, nwp):
        fn = pl.kernel(
            body,
            out_type=jax.ShapeDtypeStruct((2, nck, NP, f), jnp.float32),
            mesh=_MESH,
            compiler_params=_SC_PARAMS,
            scratch_types=[
                pltpu.VMEM((NBLKH, BH), jnp.int32),
                pltpu.VMEM((NBLKH, BH), jnp.int32),
                pltpu.VMEM((NBLKH, BH), jnp.float32),
                pltpu.VMEM((BH, f), jnp.float32),
                pltpu.VMEM_SHARED((NP, f), jnp.float32),
            ],
        )
        return fn(h2, srcp.reshape(NW, NBLKH, BH),
                  dstp.reshape(NW, NBLKH, BH), nwp.reshape(NW, NBLKH, BH))

    return call


_hop_l1 = _make_hop(2, 128)
_hop_l2 = _make_hop(1, 128)


def _finalize_body(p_ref, invs_ref, selfw_ref):
    d = 1.0 + p_ref[0] + p_ref[1]
    d = jnp.maximum(d, 1e-12)
    invs_ref[...] = lax.rsqrt(d)[:, :1]
    selfw_ref[...] = (1.0 / d)[:, :1]


def _finalize(dpart):
    return pl.pallas_call(
        _finalize_body,
        grid=(NP // ROW_BLK,),
        in_specs=[pl.BlockSpec((2, ROW_BLK, L), lambda i: (0, i, 0))],
        out_specs=[pl.BlockSpec((ROW_BLK, 1), lambda i: (i, 0)),
                   pl.BlockSpec((ROW_BLK, 1), lambda i: (i, 0))],
        out_shape=[jax.ShapeDtypeStruct((NP, 1), jnp.float32),
                   jax.ShapeDtypeStruct((NP, 1), jnp.float32)],
    )(dpart)


def _combine_body(nck, p_ref, h_ref, sw_ref, o_ref):
    sw = sw_ref[...]
    for ck in range(nck):
        o_ref[ck] = p_ref[0, ck] + p_ref[1, ck] + sw * h_ref[ck]


def _combine(parts, h2, selfw):
    nck, _, f = h2.shape
    return pl.pallas_call(
        functools.partial(_combine_body, nck),
        grid=(NP // ROW_BLK,),
        in_specs=[
            pl.BlockSpec((2, nck, ROW_BLK, f), lambda i: (0, 0, i, 0)),
            pl.BlockSpec((nck, ROW_BLK, f), lambda i: (0, i, 0)),
            pl.BlockSpec((ROW_BLK, 1), lambda i: (i, 0)),
        ],
        out_specs=pl.BlockSpec((nck, ROW_BLK, f), lambda i: (0, i, 0)),
        out_shape=jax.ShapeDtypeStruct((nck, NP, f), jnp.float32),
    )(parts, h2, selfw)


def _mm_body(nck, relu, h0_ref, h1_ref, h2_ref, h3_ref, w_ref, b_ref, o_ref):
    o = o_ref.shape[-1]
    acc = jnp.zeros((ROW_BLK, o), jnp.float32) + b_ref[...][None, :]
    for k, href in enumerate((h0_ref, h1_ref, h2_ref, h3_ref)):
        for ck in range(nck):
            acc += jnp.dot(href[ck], w_ref[k, ck],
                           preferred_element_type=jnp.float32)
    if relu:
        acc = jnp.maximum(acc, 0.0)
    o_ref[...] = acc


def _mm(hops, w, b, relu):
    nck, _, f = hops[0].shape
    o = w.shape[-1]
    hspec = pl.BlockSpec((nck, ROW_BLK, f), lambda i: (0, i, 0))
    return pl.pallas_call(
        functools.partial(_mm_body, nck, relu),
        grid=(NP // ROW_BLK,),
        in_specs=[hspec, hspec, hspec, hspec,
                  pl.BlockSpec((4, nck, f, o), lambda i: (0, 0, 0, 0)),
                  pl.BlockSpec((o,), lambda i: (0,))],
        out_specs=pl.BlockSpec((ROW_BLK, o), lambda i: (i, 0)),
        out_shape=jax.ShapeDtypeStruct((NP, o), jnp.float32),
    )(*hops, w, b)


def kernel(x, edge_index, edge_weight, W0, b0, W1, b1):
    src = edge_index[0]
    dst = edge_index[1]
    pad = EP - E
    srcp = jnp.pad(src, (0, pad)).reshape(NW, NBLK, B)
    dstp = jnp.pad(dst, (0, pad)).reshape(NW, NBLK, B)
    wp = jnp.pad(edge_weight, (0, pad)).reshape(NW, NBLK, B)

    dpart = _deg_call(dstp, wp)
    invs1, selfw = _finalize(dpart)
    nwp = _normw_call(invs1.reshape(NP), srcp, dstp, wp)

    xp = jnp.pad(x, ((0, NP - N), (0, 0)))
    h2 = xp.reshape(NP, 2, 128).transpose(1, 0, 2)
    hops1 = [h2]
    for _ in range(K):
        parts = _hop_l1(h2, srcp, dstp, nwp)
        h2 = _combine(parts, h2, selfw)
        hops1.append(h2)
    h1 = _mm(hops1, W0.reshape(K + 1, 2, 128, H), b0, relu=True)

    g = jnp.pad(h1, ((0, 0), (0, 128 - H))).reshape(1, NP, 128)
    hops2 = [g]
    for _ in range(K):
        parts = _hop_l2(g, srcp, dstp, nwp)
        g = _combine(parts, g, selfw)
        hops2.append(g)
    w1p = jnp.pad(W1.reshape(K + 1, H, C), ((0, 0), (0, 128 - H), (0, 0)))
    out = _mm(hops2, w1p.reshape(K + 1, 1, 128, C), b1, relu=False)
    return out[:N]
